# baseline (speedup 1.0000x reference)
import functools

import jax
import jax.numpy as jnp
from jax import lax
from jax.experimental import pallas as pl
from jax.experimental.pallas import tpu as pltpu
from jax.experimental.pallas import tpu_sc as plsc

_NC = 2
_NS = 16
_NW = _NC * _NS
_LANES = 16

_K = 80
_NCH = 25
_CH = _K * _NCH


def _prep_call(n_nodes, n_rels, n_edges):
    e_w = n_edges // _NW
    e_t = n_edges // _NS
    nrc = n_nodes * n_rels
    mesh = plsc.VectorSubcoreMesh(core_axis_name="c", subcore_axis_name="s")

    @functools.partial(
        pl.kernel,
        out_type=(
            jax.ShapeDtypeStruct((n_edges,), jnp.int32),
            jax.ShapeDtypeStruct((n_edges,), jnp.float32),
        ),
        mesh=mesh,
        scratch_types=[
            pltpu.VMEM_SHARED((nrc,), jnp.float32),
            pltpu.VMEM((_CH,), jnp.int32),
            pltpu.VMEM((_CH,), jnp.int32),
            pltpu.VMEM((_CH,), jnp.int32),
            pltpu.VMEM((_NCH, _K), jnp.int32),
            pltpu.VMEM((_CH,), jnp.int32),
            pltpu.VMEM((_CH,), jnp.int32),
            pltpu.VMEM((_K,), jnp.float32),
            pltpu.VMEM((_CH,), jnp.float32),
            pltpu.VMEM((_CH,), jnp.float32),
            pltpu.SemaphoreType.DMA,
        ],
        compiler_params=pltpu.CompilerParams(needs_layout_passes=False),
    )
    def prep(src_hbm, dst_hbm, et_hbm, flat_hbm, inv_hbm,
             cnt_sh, srcb, dstb, etb, comp2, comp1, flatb, onesb, valsb,
             invb, sem):
        si = lax.axis_index("s")
        ci = lax.axis_index("c")
        wid = ci * _NS + si

        def fill_ones(i, _):
            onesb[pl.ds(i * _LANES, _LANES)] = jnp.full((_LANES,), 1.0,
                                                        jnp.float32)
            return _
        lax.fori_loop(0, _K // _LANES, fill_ones, None)

        def fill_zero(i, _):
            valsb[pl.ds(i * _LANES, _LANES)] = jnp.zeros((_LANES,),
                                                         jnp.float32)
            return _
        lax.fori_loop(0, _CH // _LANES, fill_zero, None)

        def zero_chunk(c, _):
            @pl.when(c % _NS == si)
            def _do():
                pltpu.sync_copy(valsb, cnt_sh.at[pl.ds(c * _CH, _CH)])
            return _
        lax.fori_loop(0, nrc // _CH, zero_chunk, None)
        plsc.subcore_barrier()

        def count_chunk(c, _):
            base = si * e_t + c * _CH
            pltpu.sync_copy(dst_hbm.at[pl.ds(base, _CH)], dstb)
            pltpu.sync_copy(et_hbm.at[pl.ds(base, _CH)], etb)

            def comp_body(k, _c):
                def comp_grp(g, _g):
                    sl = pl.ds(k * _K + g * _LANES, _LANES)
                    v = dstb[sl] * n_rels + etb[sl]
                    comp2[k, pl.ds(g * _LANES, _LANES)] = v
                    return _g
                lax.fori_loop(0, _K // _LANES, comp_grp, None)
                pltpu.sync_copy(onesb, cnt_sh.at[comp2.at[k]], add=True)
                return _c
            lax.fori_loop(0, _NCH, comp_body, None)
            return _
        lax.fori_loop(0, e_t // _CH, count_chunk, None)
        plsc.subcore_barrier()

        def out_chunk(c, _):
            base = wid * e_w + c * _CH
            pltpu.sync_copy(src_hbm.at[pl.ds(base, _CH)], srcb)
            pltpu.sync_copy(dst_hbm.at[pl.ds(base, _CH)], dstb)
            pltpu.sync_copy(et_hbm.at[pl.ds(base, _CH)], etb)

            def idx_body(i, _i):
                sl = pl.ds(i * _LANES, _LANES)
                comp1[sl] = dstb[sl] * n_rels + etb[sl]
                flatb[sl] = etb[sl] * n_nodes + srcb[sl]
                return _i
            lax.fori_loop(0, _CH // _LANES, idx_body, None, unroll=4)

            def gather_body(k, _k):
                sl = pl.ds(k * _K, _K)
                pltpu.async_copy(cnt_sh.at[comp1.at[sl]], valsb.at[sl],
                                 sem).wait()
                return _k
            lax.fori_loop(0, _NCH, gather_body, None)

            def inv_body(i, _i):
                sl = pl.ds(i * _LANES, _LANES)
                invb[sl] = 1.0 / jnp.maximum(valsb[sl], 1.0)
                return _i
            lax.fori_loop(0, _CH // _LANES, inv_body, None, unroll=4)

            pltpu.sync_copy(flatb, flat_hbm.at[pl.ds(base, _CH)])
            pltpu.sync_copy(invb, inv_hbm.at[pl.ds(base, _CH)])
            return _
        lax.fori_loop(0, e_w // _CH, out_chunk, None)

    return prep


def _edge_call(n_nodes, d, n_edges):
    ks = 80
    ch = 2560
    nsub = ch // ks
    nbuf = 3
    e_w = n_edges // _NW
    nsl = (n_nodes // _NS) // 8 * 8
    tail = n_nodes - _NS * nsl
    mesh = plsc.VectorSubcoreMesh(core_axis_name="c", subcore_axis_name="s")

    @functools.partial(
        pl.kernel,
        out_type=jax.ShapeDtypeStruct((_NC * n_nodes, d), jnp.float32),
        mesh=mesh,
        scratch_types=[
            pltpu.VMEM_SHARED((n_nodes, d), jnp.float32),
            pltpu.VMEM((ch,), jnp.int32),
            pltpu.VMEM((nsub, ks), jnp.int32),
            pltpu.VMEM((ch,), jnp.float32),
        ] + [pltpu.VMEM((ks, d), jnp.float32) for _ in range(nbuf)]
          + [pltpu.SemaphoreType.DMA for _ in range(2 * nbuf)],
        compiler_params=pltpu.CompilerParams(needs_layout_passes=False),
    )
    def edge(table_hbm, flat_hbm, dst_hbm, inv_hbm, zero_hbm, out_hbm,
             agg_sh, idxb, dst2, invb, *bufs_and_sems):
        si = lax.axis_index("s")
        ci = lax.axis_index("c")
        wid = ci * _NS + si
        rowsb = bufs_and_sems[:nbuf]
        gsem = bufs_and_sems[nbuf:2 * nbuf]
        ssem = bufs_and_sems[2 * nbuf:3 * nbuf]

        pltpu.sync_copy(zero_hbm.at[pl.ds(si * nsl, nsl)],
                        agg_sh.at[pl.ds(si * nsl, nsl)])
        if tail:
            @pl.when(si == _NS - 1)
            def _init_tail():
                pltpu.sync_copy(zero_hbm.at[pl.ds(_NS * nsl, tail)],
                                agg_sh.at[pl.ds(_NS * nsl, tail)])
        plsc.subcore_barrier()

        def scale(rb, off):
            def edge_body(e, _e):
                iv = plsc.load_gather(
                    invb, [jnp.full((_LANES,), off + e, jnp.int32)])
                for cc in range(d // _LANES):
                    sl = pl.ds(cc * _LANES, _LANES)
                    rb[e, sl] = rb[e, sl] * iv
                return _e
            lax.fori_loop(0, ks, edge_body, None, unroll=2)

        def chunk_body(c, _):
            base = wid * e_w + c * ch
            pltpu.sync_copy(flat_hbm.at[pl.ds(base, ch)], idxb)
            pltpu.sync_copy(inv_hbm.at[pl.ds(base, ch)], invb)
            pltpu.sync_copy(
                dst_hbm.at[pl.ds(pl.multiple_of(base // ks, 8), nsub)], dst2)

            gd = [None] * nbuf
            sd = [None] * nbuf
            for k in range(nbuf - 1):
                gd[k] = pltpu.async_copy(
                    table_hbm.at[idxb.at[pl.ds(k * ks, ks)]],
                    rowsb[k], gsem[k])
            for k in range(nsub):
                b = k % nbuf
                if k + nbuf - 1 < nsub:
                    nb = (k + nbuf - 1) % nbuf
                    if sd[nb] is not None:
                        sd[nb].wait()
                        sd[nb] = None
                    gd[nb] = pltpu.async_copy(
                        table_hbm.at[idxb.at[pl.ds((k + nbuf - 1) * ks, ks)]],
                        rowsb[nb], gsem[nb])
                gd[b].wait()
                scale(rowsb[b], k * ks)
                sd[b] = pltpu.async_copy(rowsb[b], agg_sh.at[dst2.at[k]],
                                         ssem[b], add=True)
            for b in range(nbuf):
                if sd[b] is not None:
                    sd[b].wait()
            return _
        lax.fori_loop(0, e_w // ch, chunk_body, None)

        plsc.subcore_barrier()
        pltpu.sync_copy(agg_sh.at[pl.ds(si * nsl, nsl)],
                        out_hbm.at[pl.ds(ci * n_nodes + si * nsl, nsl)])
        if tail:
            @pl.when(si == _NS - 1)
            def _out_tail():
                pltpu.sync_copy(
                    agg_sh.at[pl.ds(_NS * nsl, tail)],
                    out_hbm.at[pl.ds(ci * n_nodes + _NS * nsl, tail)])

    return edge


def _mm_call(n_nodes, d, h, n_rels, fuse_agg):
    bn = 1000
    grid = (n_nodes // bn,)

    def body(*refs):
        if fuse_agg:
            a_ref, zin_ref, w_ref, b_ref, xt_ref, z_ref = refs
            t = jax.nn.relu(a_ref[0] + a_ref[1] + zin_ref[...])
        else:
            x_ref, w_ref, b_ref, xt_ref, z_ref = refs
            t = x_ref[...]
        for r in range(n_rels):
            xt_ref[r] = jnp.dot(t, w_ref[r], preferred_element_type=jnp.float32)
        z_ref[...] = (jnp.dot(t, w_ref[n_rels],
                              preferred_element_type=jnp.float32)
                      + b_ref[...])

    in_specs = []
    if fuse_agg:
        in_specs.append(pl.BlockSpec((_NC, bn, d), lambda i: (0, i, 0)))
        in_specs.append(pl.BlockSpec((bn, d), lambda i: (i, 0)))
    else:
        in_specs.append(pl.BlockSpec((bn, d), lambda i: (i, 0)))
    in_specs.append(pl.BlockSpec((n_rels + 1, d, h), lambda i: (0, 0, 0)))
    in_specs.append(pl.BlockSpec((1, h), lambda i: (0, 0)))

    return pl.pallas_call(
        body,
        grid=grid,
        in_specs=in_specs,
        out_specs=[
            pl.BlockSpec((n_rels, bn, h), lambda i: (0, i, 0)),
            pl.BlockSpec((bn, h), lambda i: (i, 0)),
        ],
        out_shape=[
            jax.ShapeDtypeStruct((n_rels, n_nodes, h), jnp.float32),
            jax.ShapeDtypeStruct((n_nodes, h), jnp.float32),
        ],
    )


def _final_call(n_nodes, d):
    bn = 1000
    grid = (n_nodes // bn,)

    def body(a_ref, z_ref, o_ref):
        o_ref[...] = a_ref[0] + a_ref[1] + z_ref[...]

    return pl.pallas_call(
        body,
        grid=grid,
        in_specs=[
            pl.BlockSpec((_NC, bn, d), lambda i: (0, i, 0)),
            pl.BlockSpec((bn, d), lambda i: (i, 0)),
        ],
        out_specs=pl.BlockSpec((bn, d), lambda i: (i, 0)),
        out_shape=jax.ShapeDtypeStruct((n_nodes, d), jnp.float32),
    )


def kernel(x, edge_index, edge_type, W1, root1, b1, W2, root2, b2):
    n_nodes, d = x.shape
    n_rels, _, h = W1.shape
    n_edges = edge_type.shape[0]
    o = W2.shape[2]

    src = edge_index[0]
    dst = edge_index[1]
    zeros_nd = jnp.zeros((n_nodes, d), jnp.float32)

    flat_idx, inv_d = _prep_call(n_nodes, n_rels, n_edges)(src, dst, edge_type)

    ch_total = _NW * 2560
    e_pad = -(-n_edges // ch_total) * ch_total
    pad = e_pad - n_edges
    flat_p = jnp.concatenate([flat_idx, jnp.zeros((pad,), jnp.int32)])
    inv_p = jnp.concatenate([inv_d, jnp.zeros((pad,), jnp.float32)])
    dst_p = jnp.concatenate([dst, jnp.zeros((pad,), jnp.int32)]).reshape(-1, 80)

    w1c = jnp.concatenate([W1, root1[None]], axis=0)
    w2c = jnp.concatenate([W2, root2[None]], axis=0)

    xt1, z1 = _mm_call(n_nodes, d, h, n_rels, fuse_agg=False)(
        x, w1c, b1.reshape(1, h))
    agg1 = _edge_call(n_nodes, h, e_pad)(
        xt1.reshape(n_rels * n_nodes, h), flat_p, dst_p, inv_p, zeros_nd)

    xt2, z2 = _mm_call(n_nodes, h, o, n_rels, fuse_agg=True)(
        agg1.reshape(_NC, n_nodes, h), z1, w2c, b2.reshape(1, o))
    agg2 = _edge_call(n_nodes, o, e_pad)(
        xt2.reshape(n_rels * n_nodes, o), flat_p, dst_p, inv_p, zeros_nd)

    return _final_call(n_nodes, o)(agg2.reshape(_NC, n_nodes, o), z2)


# speedup vs baseline: 1.9534x; 1.9534x over previous
import functools

import jax
import jax.numpy as jnp
from jax import lax
from jax.experimental import pallas as pl
from jax.experimental.pallas import tpu as pltpu
from jax.experimental.pallas import tpu_sc as plsc

_NC = 2
_NS = 16
_NW = _NC * _NS
_LANES = 16

_K = 80
_NCH = 25
_CH = _K * _NCH


def _prep_call(n_nodes, n_rels, n_edges):
    e_w = n_edges // _NW
    e_t = n_edges // _NS
    nrc = n_nodes * n_rels
    mesh = plsc.VectorSubcoreMesh(core_axis_name="c", subcore_axis_name="s")

    @functools.partial(
        pl.kernel,
        out_type=(
            jax.ShapeDtypeStruct((n_edges,), jnp.int32),
            jax.ShapeDtypeStruct((n_edges,), jnp.float32),
        ),
        mesh=mesh,
        scratch_types=[
            pltpu.VMEM_SHARED((nrc,), jnp.float32),
            pltpu.VMEM((_CH,), jnp.int32),
            pltpu.VMEM((_CH,), jnp.int32),
            pltpu.VMEM((_CH,), jnp.int32),
            pltpu.VMEM((_NCH, _K), jnp.int32),
            pltpu.VMEM((_CH,), jnp.int32),
            pltpu.VMEM((_CH,), jnp.int32),
            pltpu.VMEM((_K,), jnp.float32),
            pltpu.VMEM((_CH,), jnp.float32),
            pltpu.VMEM((_CH,), jnp.float32),
            pltpu.SemaphoreType.DMA,
        ],
        compiler_params=pltpu.CompilerParams(needs_layout_passes=False),
    )
    def prep(src_hbm, dst_hbm, et_hbm, flat_hbm, inv_hbm,
             cnt_sh, srcb, dstb, etb, comp2, comp1, flatb, onesb, valsb,
             invb, sem):
        si = lax.axis_index("s")
        ci = lax.axis_index("c")
        wid = ci * _NS + si

        def fill_ones(i, _):
            onesb[pl.ds(i * _LANES, _LANES)] = jnp.full((_LANES,), 1.0,
                                                        jnp.float32)
            return _
        lax.fori_loop(0, _K // _LANES, fill_ones, None)

        def fill_zero(i, _):
            valsb[pl.ds(i * _LANES, _LANES)] = jnp.zeros((_LANES,),
                                                         jnp.float32)
            return _
        lax.fori_loop(0, _CH // _LANES, fill_zero, None)

        def zero_chunk(c, _):
            @pl.when(c % _NS == si)
            def _do():
                pltpu.sync_copy(valsb, cnt_sh.at[pl.ds(c * _CH, _CH)])
            return _
        lax.fori_loop(0, nrc // _CH, zero_chunk, None)
        plsc.subcore_barrier()

        def count_chunk(c, _):
            base = si * e_t + c * _CH
            pltpu.sync_copy(dst_hbm.at[pl.ds(base, _CH)], dstb)
            pltpu.sync_copy(et_hbm.at[pl.ds(base, _CH)], etb)

            def comp_body(k, _c):
                def comp_grp(g, _g):
                    sl = pl.ds(k * _K + g * _LANES, _LANES)
                    v = dstb[sl] * n_rels + etb[sl]
                    comp2[k, pl.ds(g * _LANES, _LANES)] = v
                    return _g
                lax.fori_loop(0, _K // _LANES, comp_grp, None)
                pltpu.sync_copy(onesb, cnt_sh.at[comp2.at[k]], add=True)
                return _c
            lax.fori_loop(0, _NCH, comp_body, None)
            return _
        lax.fori_loop(0, e_t // _CH, count_chunk, None)
        plsc.subcore_barrier()

        def out_chunk(c, _):
            base = wid * e_w + c * _CH
            pltpu.sync_copy(src_hbm.at[pl.ds(base, _CH)], srcb)
            pltpu.sync_copy(dst_hbm.at[pl.ds(base, _CH)], dstb)
            pltpu.sync_copy(et_hbm.at[pl.ds(base, _CH)], etb)

            def idx_body(i, _i):
                sl = pl.ds(i * _LANES, _LANES)
                comp1[sl] = dstb[sl] * n_rels + etb[sl]
                flatb[sl] = etb[sl] * n_nodes + srcb[sl]
                return _i
            lax.fori_loop(0, _CH // _LANES, idx_body, None, unroll=4)

            def gather_body(k, _k):
                sl = pl.ds(k * _K, _K)
                pltpu.async_copy(cnt_sh.at[comp1.at[sl]], valsb.at[sl],
                                 sem).wait()
                return _k
            lax.fori_loop(0, _NCH, gather_body, None)

            def inv_body(i, _i):
                sl = pl.ds(i * _LANES, _LANES)
                invb[sl] = 1.0 / jnp.maximum(valsb[sl], 1.0)
                return _i
            lax.fori_loop(0, _CH // _LANES, inv_body, None, unroll=4)

            pltpu.sync_copy(flatb, flat_hbm.at[pl.ds(base, _CH)])
            pltpu.sync_copy(invb, inv_hbm.at[pl.ds(base, _CH)])
            return _
        lax.fori_loop(0, e_w // _CH, out_chunk, None)

    return prep


def _edge_call(n_nodes, d, n_edges):
    ks = 80
    ch = 2560
    nsub = ch // ks
    nbuf = 3
    e_w = n_edges // _NW
    nsl = (n_nodes // _NS) // 8 * 8
    tail = n_nodes - _NS * nsl
    mesh = plsc.VectorSubcoreMesh(core_axis_name="c", subcore_axis_name="s")

    @functools.partial(
        pl.kernel,
        out_type=jax.ShapeDtypeStruct((_NC * n_nodes, d), jnp.float32),
        mesh=mesh,
        scratch_types=[
            pltpu.VMEM_SHARED((n_nodes, d), jnp.float32),
            pltpu.VMEM((ch,), jnp.int32),
            pltpu.VMEM((nsub, ks), jnp.int32),
            pltpu.VMEM((ch,), jnp.float32),
        ] + [pltpu.VMEM((ks, d), jnp.float32) for _ in range(nbuf)]
          + [pltpu.SemaphoreType.DMA for _ in range(2 * nbuf)],
        compiler_params=pltpu.CompilerParams(needs_layout_passes=False),
    )
    def edge(table_hbm, flat_hbm, dst_hbm, inv_hbm, zero_hbm, out_hbm,
             agg_sh, idxb, dst2, invb, *bufs_and_sems):
        si = lax.axis_index("s")
        ci = lax.axis_index("c")
        wid = ci * _NS + si
        rowsb = bufs_and_sems[:nbuf]
        gsem = bufs_and_sems[nbuf:2 * nbuf]
        ssem = bufs_and_sems[2 * nbuf:3 * nbuf]

        pltpu.sync_copy(zero_hbm.at[pl.ds(si * nsl, nsl)],
                        agg_sh.at[pl.ds(si * nsl, nsl)])
        if tail:
            @pl.when(si == _NS - 1)
            def _init_tail():
                pltpu.sync_copy(zero_hbm.at[pl.ds(_NS * nsl, tail)],
                                agg_sh.at[pl.ds(_NS * nsl, tail)])
        plsc.subcore_barrier()

        def scale(rb, off):
            def edge_body(e, _e):
                iv = plsc.load_gather(
                    invb, [jnp.full((_LANES,), off + e, jnp.int32)])
                for cc in range(d // _LANES):
                    sl = pl.ds(cc * _LANES, _LANES)
                    rb[e, sl] = rb[e, sl] * iv
                return _e
            lax.fori_loop(0, ks, edge_body, None, unroll=2)

        def chunk_body(c, _):
            base = wid * e_w + c * ch
            pltpu.sync_copy(flat_hbm.at[pl.ds(base, ch)], idxb)
            pltpu.sync_copy(inv_hbm.at[pl.ds(base, ch)], invb)
            pltpu.sync_copy(
                dst_hbm.at[pl.ds(pl.multiple_of(base // ks, 8), nsub)], dst2)

            gd = [None] * nbuf
            sd = [None] * nbuf
            for k in range(nbuf - 1):
                gd[k] = pltpu.async_copy(
                    table_hbm.at[idxb.at[pl.ds(k * ks, ks)]],
                    rowsb[k], gsem[k])
            for k in range(nsub):
                b = k % nbuf
                if k + nbuf - 1 < nsub:
                    nb = (k + nbuf - 1) % nbuf
                    if sd[nb] is not None:
                        sd[nb].wait()
                        sd[nb] = None
                    gd[nb] = pltpu.async_copy(
                        table_hbm.at[idxb.at[pl.ds((k + nbuf - 1) * ks, ks)]],
                        rowsb[nb], gsem[nb])
                gd[b].wait()
                scale(rowsb[b], k * ks)
                sd[b] = pltpu.async_copy(rowsb[b], agg_sh.at[dst2.at[k]],
                                         ssem[b], add=True)
            for b in range(nbuf):
                if sd[b] is not None:
                    sd[b].wait()
            return _
        lax.fori_loop(0, e_w // ch, chunk_body, None)

        plsc.subcore_barrier()
        pltpu.sync_copy(agg_sh.at[pl.ds(si * nsl, nsl)],
                        out_hbm.at[pl.ds(ci * n_nodes + si * nsl, nsl)])
        if tail:
            @pl.when(si == _NS - 1)
            def _out_tail():
                pltpu.sync_copy(
                    agg_sh.at[pl.ds(_NS * nsl, tail)],
                    out_hbm.at[pl.ds(ci * n_nodes + _NS * nsl, tail)])

    return edge


def _mm_call(n_nodes, d, h, n_rels, fuse_agg):
    bn = 1000
    grid = (n_nodes // bn,)

    def body(*refs):
        if fuse_agg:
            a_ref, zin_ref, w_ref, b_ref, xt_ref, z_ref = refs
            t = jax.nn.relu(a_ref[0] + a_ref[1] + zin_ref[...])
        else:
            x_ref, w_ref, b_ref, xt_ref, z_ref = refs
            t = x_ref[...]
        for r in range(n_rels):
            xt_ref[r] = jnp.dot(t, w_ref[r], preferred_element_type=jnp.float32)
        z_ref[...] = (jnp.dot(t, w_ref[n_rels],
                              preferred_element_type=jnp.float32)
                      + b_ref[...])

    in_specs = []
    if fuse_agg:
        in_specs.append(pl.BlockSpec((_NC, bn, d), lambda i: (0, i, 0)))
        in_specs.append(pl.BlockSpec((bn, d), lambda i: (i, 0)))
    else:
        in_specs.append(pl.BlockSpec((bn, d), lambda i: (i, 0)))
    in_specs.append(pl.BlockSpec((n_rels + 1, d, h), lambda i: (0, 0, 0)))
    in_specs.append(pl.BlockSpec((1, h), lambda i: (0, 0)))

    return pl.pallas_call(
        body,
        grid=grid,
        in_specs=in_specs,
        out_specs=[
            pl.BlockSpec((n_rels, bn, h), lambda i: (0, i, 0)),
            pl.BlockSpec((bn, h), lambda i: (i, 0)),
        ],
        out_shape=[
            jax.ShapeDtypeStruct((n_rels, n_nodes, h), jnp.float32),
            jax.ShapeDtypeStruct((n_nodes, h), jnp.float32),
        ],
    )


def _final_call(n_nodes, d):
    bn = 1000
    grid = (n_nodes // bn,)

    def body(a_ref, z_ref, o_ref):
        o_ref[...] = a_ref[0] + a_ref[1] + z_ref[...]

    return pl.pallas_call(
        body,
        grid=grid,
        in_specs=[
            pl.BlockSpec((_NC, bn, d), lambda i: (0, i, 0)),
            pl.BlockSpec((bn, d), lambda i: (i, 0)),
        ],
        out_specs=pl.BlockSpec((bn, d), lambda i: (i, 0)),
        out_shape=jax.ShapeDtypeStruct((n_nodes, d), jnp.float32),
    )


def kernel(x, edge_index, edge_type, W1, root1, b1, W2, root2, b2):
    n_nodes, d = x.shape
    n_rels, _, h = W1.shape
    n_edges = edge_type.shape[0]
    o = W2.shape[2]

    src = edge_index[0]
    dst = edge_index[1]
    zeros_nd = jnp.zeros((n_nodes, d), jnp.float32)

    flat_idx, inv_d = _prep_call(n_nodes, n_rels, n_edges)(src, dst, edge_type)

    ch_total = _NW * 2560
    e_pad = -(-n_edges // ch_total) * ch_total
    pad = e_pad - n_edges
    spread = jnp.arange(pad, dtype=jnp.int32) % n_nodes
    flat_p = jnp.concatenate([flat_idx, spread])
    inv_p = jnp.concatenate([inv_d, jnp.zeros((pad,), jnp.float32)])
    dst_p = jnp.concatenate([dst, spread]).reshape(-1, 80)

    w1c = jnp.concatenate([W1, root1[None]], axis=0)
    w2c = jnp.concatenate([W2, root2[None]], axis=0)

    xt1, z1 = _mm_call(n_nodes, d, h, n_rels, fuse_agg=False)(
        x, w1c, b1.reshape(1, h))
    agg1 = _edge_call(n_nodes, h, e_pad)(
        xt1.reshape(n_rels * n_nodes, h), flat_p, dst_p, inv_p, zeros_nd)

    xt2, z2 = _mm_call(n_nodes, h, o, n_rels, fuse_agg=True)(
        agg1.reshape(_NC, n_nodes, h), z1, w2c, b2.reshape(1, o))
    agg2 = _edge_call(n_nodes, o, e_pad)(
        xt2.reshape(n_rels * n_nodes, o), flat_p, dst_p, inv_p, zeros_nd)

    return _final_call(n_nodes, o)(agg2.reshape(_NC, n_nodes, o), z2)


# speedup vs baseline: 1.9819x; 1.0146x over previous
import functools

import jax
import jax.numpy as jnp
from jax import lax
from jax.experimental import pallas as pl
from jax.experimental.pallas import tpu as pltpu
from jax.experimental.pallas import tpu_sc as plsc

_NC = 2
_NS = 16
_NW = _NC * _NS
_LANES = 16

_K = 80
_NCH = 25
_CH = _K * _NCH


def _prep_call(n_nodes, n_rels, n_edges):
    e_w = n_edges // _NW
    e_t = n_edges // _NS
    nrc = n_nodes * n_rels
    mesh = plsc.VectorSubcoreMesh(core_axis_name="c", subcore_axis_name="s")

    @functools.partial(
        pl.kernel,
        out_type=(
            jax.ShapeDtypeStruct((n_edges,), jnp.int32),
            jax.ShapeDtypeStruct((n_edges,), jnp.float32),
        ),
        mesh=mesh,
        scratch_types=[
            pltpu.VMEM_SHARED((nrc,), jnp.float32),
            pltpu.VMEM((_CH,), jnp.int32),
            pltpu.VMEM((_CH,), jnp.int32),
            pltpu.VMEM((_CH,), jnp.int32),
            pltpu.VMEM((_NCH, _K), jnp.int32),
            pltpu.VMEM((_CH,), jnp.int32),
            pltpu.VMEM((_CH,), jnp.int32),
            pltpu.VMEM((_K,), jnp.float32),
            pltpu.VMEM((_CH,), jnp.float32),
            pltpu.VMEM((_CH,), jnp.float32),
            pltpu.SemaphoreType.DMA,
        ],
        compiler_params=pltpu.CompilerParams(needs_layout_passes=False),
    )
    def prep(src_hbm, dst_hbm, et_hbm, flat_hbm, inv_hbm,
             cnt_sh, srcb, dstb, etb, comp2, comp1, flatb, onesb, valsb,
             invb, sem):
        si = lax.axis_index("s")
        ci = lax.axis_index("c")
        wid = ci * _NS + si

        def fill_ones(i, _):
            onesb[pl.ds(i * _LANES, _LANES)] = jnp.full((_LANES,), 1.0,
                                                        jnp.float32)
            return _
        lax.fori_loop(0, _K // _LANES, fill_ones, None)

        def fill_zero(i, _):
            valsb[pl.ds(i * _LANES, _LANES)] = jnp.zeros((_LANES,),
                                                         jnp.float32)
            return _
        lax.fori_loop(0, _CH // _LANES, fill_zero, None)

        def zero_chunk(c, _):
            @pl.when(c % _NS == si)
            def _do():
                pltpu.sync_copy(valsb, cnt_sh.at[pl.ds(c * _CH, _CH)])
            return _
        lax.fori_loop(0, nrc // _CH, zero_chunk, None)
        plsc.subcore_barrier()

        def count_chunk(c, _):
            base = si * e_t + c * _CH
            pltpu.sync_copy(dst_hbm.at[pl.ds(base, _CH)], dstb)
            pltpu.sync_copy(et_hbm.at[pl.ds(base, _CH)], etb)

            def comp_body(k, _c):
                def comp_grp(g, _g):
                    sl = pl.ds(k * _K + g * _LANES, _LANES)
                    v = dstb[sl] * n_rels + etb[sl]
                    comp2[k, pl.ds(g * _LANES, _LANES)] = v
                    return _g
                lax.fori_loop(0, _K // _LANES, comp_grp, None)
                pltpu.sync_copy(onesb, cnt_sh.at[comp2.at[k]], add=True)
                return _c
            lax.fori_loop(0, _NCH, comp_body, None)
            return _
        lax.fori_loop(0, e_t // _CH, count_chunk, None)
        plsc.subcore_barrier()

        def out_chunk(c, _):
            base = wid * e_w + c * _CH
            pltpu.sync_copy(src_hbm.at[pl.ds(base, _CH)], srcb)
            pltpu.sync_copy(dst_hbm.at[pl.ds(base, _CH)], dstb)
            pltpu.sync_copy(et_hbm.at[pl.ds(base, _CH)], etb)

            def idx_body(i, _i):
                sl = pl.ds(i * _LANES, _LANES)
                comp1[sl] = dstb[sl] * n_rels + etb[sl]
                flatb[sl] = etb[sl] * n_nodes + srcb[sl]
                return _i
            lax.fori_loop(0, _CH // _LANES, idx_body, None, unroll=4)

            def gather_body(k, _k):
                sl = pl.ds(k * _K, _K)
                pltpu.async_copy(cnt_sh.at[comp1.at[sl]], valsb.at[sl],
                                 sem).wait()
                return _k
            lax.fori_loop(0, _NCH, gather_body, None)

            def inv_body(i, _i):
                sl = pl.ds(i * _LANES, _LANES)
                invb[sl] = 1.0 / jnp.maximum(valsb[sl], 1.0)
                return _i
            lax.fori_loop(0, _CH // _LANES, inv_body, None, unroll=4)

            pltpu.sync_copy(flatb, flat_hbm.at[pl.ds(base, _CH)])
            pltpu.sync_copy(invb, inv_hbm.at[pl.ds(base, _CH)])
            return _
        lax.fori_loop(0, e_w // _CH, out_chunk, None)

    return prep


def _edge_call(n_nodes, d, n_edges):
    ks = 80
    ch = 2560
    nsub = ch // ks
    nbuf = 4
    e_w = n_edges // _NW
    nsl = (n_nodes // _NS) // 8 * 8
    tail = n_nodes - _NS * nsl
    mesh = plsc.VectorSubcoreMesh(core_axis_name="c", subcore_axis_name="s")

    @functools.partial(
        pl.kernel,
        out_type=jax.ShapeDtypeStruct((_NC * n_nodes, d), jnp.float32),
        mesh=mesh,
        scratch_types=[
            pltpu.VMEM_SHARED((n_nodes, d), jnp.float32),
            pltpu.VMEM((ch,), jnp.int32),
            pltpu.VMEM((nsub, ks), jnp.int32),
            pltpu.VMEM((ch,), jnp.float32),
        ] + [pltpu.VMEM((ks, d), jnp.float32) for _ in range(nbuf)]
          + [pltpu.SemaphoreType.DMA for _ in range(2 * nbuf + 1)],
        compiler_params=pltpu.CompilerParams(needs_layout_passes=False),
    )
    def edge(table_hbm, flat_hbm, dst_hbm, inv_hbm, zero_hbm, out_hbm,
             agg_sh, idxb, dst2, invb, *bufs_and_sems):
        si = lax.axis_index("s")
        ci = lax.axis_index("c")
        wid = ci * _NS + si
        rowsb = bufs_and_sems[:nbuf]
        gsem = bufs_and_sems[nbuf:2 * nbuf]
        ssem = bufs_and_sems[2 * nbuf:3 * nbuf]
        hsem = bufs_and_sems[3 * nbuf]

        pltpu.sync_copy(zero_hbm.at[pl.ds(si * nsl, nsl)],
                        agg_sh.at[pl.ds(si * nsl, nsl)])
        if tail:
            @pl.when(si == _NS - 1)
            def _init_tail():
                pltpu.sync_copy(zero_hbm.at[pl.ds(_NS * nsl, tail)],
                                agg_sh.at[pl.ds(_NS * nsl, tail)])
        plsc.subcore_barrier()

        def scale(rb, off):
            def edge_body(e, _e):
                iv = plsc.load_gather(
                    invb, [jnp.full((_LANES,), off + e, jnp.int32)])
                for cc in range(d // _LANES):
                    sl = pl.ds(cc * _LANES, _LANES)
                    rb[e, sl] = rb[e, sl] * iv
                return _e
            lax.fori_loop(0, ks, edge_body, None, unroll=2)

        def chunk_body(c, _):
            base = wid * e_w + c * ch
            h0 = pltpu.async_copy(flat_hbm.at[pl.ds(base, ch)], idxb, hsem)
            h1 = pltpu.async_copy(inv_hbm.at[pl.ds(base, ch)], invb, hsem)
            h2 = pltpu.async_copy(
                dst_hbm.at[pl.ds(pl.multiple_of(base // ks, 8), nsub)], dst2,
                hsem)
            h0.wait()
            h1.wait()
            h2.wait()

            gd = [None] * nbuf
            sd = [None] * nbuf
            for k in range(nbuf - 1):
                gd[k] = pltpu.async_copy(
                    table_hbm.at[idxb.at[pl.ds(k * ks, ks)]],
                    rowsb[k], gsem[k])
            for k in range(nsub):
                b = k % nbuf
                if k + nbuf - 1 < nsub:
                    nb = (k + nbuf - 1) % nbuf
                    if sd[nb] is not None:
                        sd[nb].wait()
                        sd[nb] = None
                    gd[nb] = pltpu.async_copy(
                        table_hbm.at[idxb.at[pl.ds((k + nbuf - 1) * ks, ks)]],
                        rowsb[nb], gsem[nb])
                gd[b].wait()
                scale(rowsb[b], k * ks)
                sd[b] = pltpu.async_copy(rowsb[b], agg_sh.at[dst2.at[k]],
                                         ssem[b], add=True)
            for b in range(nbuf):
                if sd[b] is not None:
                    sd[b].wait()
            return _
        lax.fori_loop(0, e_w // ch, chunk_body, None)

        plsc.subcore_barrier()
        pltpu.sync_copy(agg_sh.at[pl.ds(si * nsl, nsl)],
                        out_hbm.at[pl.ds(ci * n_nodes + si * nsl, nsl)])
        if tail:
            @pl.when(si == _NS - 1)
            def _out_tail():
                pltpu.sync_copy(
                    agg_sh.at[pl.ds(_NS * nsl, tail)],
                    out_hbm.at[pl.ds(ci * n_nodes + _NS * nsl, tail)])

    return edge


def _mm_call(n_nodes, d, h, n_rels, fuse_agg):
    bn = 1000
    grid = (n_nodes // bn,)

    def body(*refs):
        if fuse_agg:
            a_ref, zin_ref, w_ref, b_ref, xt_ref, z_ref = refs
            t = jax.nn.relu(a_ref[0] + a_ref[1] + zin_ref[...])
        else:
            x_ref, w_ref, b_ref, xt_ref, z_ref = refs
            t = x_ref[...]
        for r in range(n_rels):
            xt_ref[r] = jnp.dot(t, w_ref[r], preferred_element_type=jnp.float32)
        z_ref[...] = (jnp.dot(t, w_ref[n_rels],
                              preferred_element_type=jnp.float32)
                      + b_ref[...])

    in_specs = []
    if fuse_agg:
        in_specs.append(pl.BlockSpec((_NC, bn, d), lambda i: (0, i, 0)))
        in_specs.append(pl.BlockSpec((bn, d), lambda i: (i, 0)))
    else:
        in_specs.append(pl.BlockSpec((bn, d), lambda i: (i, 0)))
    in_specs.append(pl.BlockSpec((n_rels + 1, d, h), lambda i: (0, 0, 0)))
    in_specs.append(pl.BlockSpec((1, h), lambda i: (0, 0)))

    return pl.pallas_call(
        body,
        grid=grid,
        in_specs=in_specs,
        out_specs=[
            pl.BlockSpec((n_rels, bn, h), lambda i: (0, i, 0)),
            pl.BlockSpec((bn, h), lambda i: (i, 0)),
        ],
        out_shape=[
            jax.ShapeDtypeStruct((n_rels, n_nodes, h), jnp.float32),
            jax.ShapeDtypeStruct((n_nodes, h), jnp.float32),
        ],
    )


def _final_call(n_nodes, d):
    bn = 1000
    grid = (n_nodes // bn,)

    def body(a_ref, z_ref, o_ref):
        o_ref[...] = a_ref[0] + a_ref[1] + z_ref[...]

    return pl.pallas_call(
        body,
        grid=grid,
        in_specs=[
            pl.BlockSpec((_NC, bn, d), lambda i: (0, i, 0)),
            pl.BlockSpec((bn, d), lambda i: (i, 0)),
        ],
        out_specs=pl.BlockSpec((bn, d), lambda i: (i, 0)),
        out_shape=jax.ShapeDtypeStruct((n_nodes, d), jnp.float32),
    )


def kernel(x, edge_index, edge_type, W1, root1, b1, W2, root2, b2):
    n_nodes, d = x.shape
    n_rels, _, h = W1.shape
    n_edges = edge_type.shape[0]
    o = W2.shape[2]

    src = edge_index[0]
    dst = edge_index[1]
    zeros_nd = jnp.zeros((n_nodes, d), jnp.float32)

    flat_idx, inv_d = _prep_call(n_nodes, n_rels, n_edges)(src, dst, edge_type)

    ch_total = _NW * 2560
    e_pad = -(-n_edges // ch_total) * ch_total
    pad = e_pad - n_edges
    spread = jnp.arange(pad, dtype=jnp.int32) % n_nodes
    flat_p = jnp.concatenate([flat_idx, spread])
    inv_p = jnp.concatenate([inv_d, jnp.zeros((pad,), jnp.float32)])
    dst_p = jnp.concatenate([dst, spread]).reshape(-1, 80)

    w1c = jnp.concatenate([W1, root1[None]], axis=0)
    w2c = jnp.concatenate([W2, root2[None]], axis=0)

    xt1, z1 = _mm_call(n_nodes, d, h, n_rels, fuse_agg=False)(
        x, w1c, b1.reshape(1, h))
    agg1 = _edge_call(n_nodes, h, e_pad)(
        xt1.reshape(n_rels * n_nodes, h), flat_p, dst_p, inv_p, zeros_nd)

    xt2, z2 = _mm_call(n_nodes, h, o, n_rels, fuse_agg=True)(
        agg1.reshape(_NC, n_nodes, h), z1, w2c, b2.reshape(1, o))
    agg2 = _edge_call(n_nodes, o, e_pad)(
        xt2.reshape(n_rels * n_nodes, o), flat_p, dst_p, inv_p, zeros_nd)

    return _final_call(n_nodes, o)(agg2.reshape(_NC, n_nodes, o), z2)


# speedup vs baseline: 2.0642x; 1.0415x over previous
import functools

import jax
import jax.numpy as jnp
from jax import lax
from jax.experimental import pallas as pl
from jax.experimental.pallas import tpu as pltpu
from jax.experimental.pallas import tpu_sc as plsc

_NC = 2
_NS = 16
_NW = _NC * _NS
_LANES = 16

_K = 80
_NCH = 25
_CH = _K * _NCH


def _prep_call(n_nodes, n_rels, n_edges):
    e_w = n_edges // _NW
    e_t = n_edges // _NS
    nrc = n_nodes * n_rels
    mesh = plsc.VectorSubcoreMesh(core_axis_name="c", subcore_axis_name="s")

    @functools.partial(
        pl.kernel,
        out_type=(
            jax.ShapeDtypeStruct((n_edges,), jnp.int32),
            jax.ShapeDtypeStruct((n_edges,), jnp.float32),
        ),
        mesh=mesh,
        scratch_types=[
            pltpu.VMEM_SHARED((nrc,), jnp.float32),
            pltpu.VMEM((_CH,), jnp.int32),
            pltpu.VMEM((_CH,), jnp.int32),
            pltpu.VMEM((_CH,), jnp.int32),
            pltpu.VMEM((_CH,), jnp.int32),
            pltpu.VMEM((_CH,), jnp.int32),
            pltpu.VMEM((_CH,), jnp.float32),
            pltpu.VMEM((_CH,), jnp.float32),
            pltpu.VMEM((_CH,), jnp.float32),
            pltpu.SemaphoreType.DMA,
        ],
        compiler_params=pltpu.CompilerParams(needs_layout_passes=False),
    )
    def prep(src_hbm, dst_hbm, et_hbm, flat_hbm, inv_hbm,
             cnt_sh, srcb, dstb, etb, comp1, flatb, onesb, valsb,
             invb, sem):
        si = lax.axis_index("s")
        ci = lax.axis_index("c")
        wid = ci * _NS + si

        def fill_ones(i, _):
            onesb[pl.ds(i * _LANES, _LANES)] = jnp.full((_LANES,), 1.0,
                                                        jnp.float32)
            return _
        lax.fori_loop(0, _CH // _LANES, fill_ones, None, unroll=4)

        def fill_zero(i, _):
            valsb[pl.ds(i * _LANES, _LANES)] = jnp.zeros((_LANES,),
                                                         jnp.float32)
            return _
        lax.fori_loop(0, _CH // _LANES, fill_zero, None)

        def zero_chunk(c, _):
            @pl.when(c % _NS == si)
            def _do():
                pltpu.sync_copy(valsb, cnt_sh.at[pl.ds(c * _CH, _CH)])
            return _
        lax.fori_loop(0, nrc // _CH, zero_chunk, None)
        plsc.subcore_barrier()

        def count_chunk(c, _):
            base = si * e_t + c * _CH
            pltpu.sync_copy(dst_hbm.at[pl.ds(base, _CH)], dstb)
            pltpu.sync_copy(et_hbm.at[pl.ds(base, _CH)], etb)

            def comp_grp(g, _g):
                sl = pl.ds(g * _LANES, _LANES)
                comp1[sl] = dstb[sl] * n_rels + etb[sl]
                return _g
            lax.fori_loop(0, _CH // _LANES, comp_grp, None, unroll=4)
            pltpu.sync_copy(onesb, cnt_sh.at[comp1], add=True)
            return _
        lax.fori_loop(0, e_t // _CH, count_chunk, None)
        plsc.subcore_barrier()

        def out_chunk(c, _):
            base = wid * e_w + c * _CH
            pltpu.sync_copy(src_hbm.at[pl.ds(base, _CH)], srcb)
            pltpu.sync_copy(dst_hbm.at[pl.ds(base, _CH)], dstb)
            pltpu.sync_copy(et_hbm.at[pl.ds(base, _CH)], etb)

            def idx_body(i, _i):
                sl = pl.ds(i * _LANES, _LANES)
                comp1[sl] = dstb[sl] * n_rels + etb[sl]
                flatb[sl] = etb[sl] * n_nodes + srcb[sl]
                return _i
            lax.fori_loop(0, _CH // _LANES, idx_body, None, unroll=4)

            pltpu.async_copy(cnt_sh.at[comp1], valsb, sem).wait()

            def inv_body(i, _i):
                sl = pl.ds(i * _LANES, _LANES)
                invb[sl] = 1.0 / jnp.maximum(valsb[sl], 1.0)
                return _i
            lax.fori_loop(0, _CH // _LANES, inv_body, None, unroll=4)

            pltpu.sync_copy(flatb, flat_hbm.at[pl.ds(base, _CH)])
            pltpu.sync_copy(invb, inv_hbm.at[pl.ds(base, _CH)])
            return _
        lax.fori_loop(0, e_w // _CH, out_chunk, None)

    return prep


def _edge_call(n_nodes, d, n_edges):
    ks = 80
    ch = 2560
    nsub = ch // ks
    nbuf = 4
    e_w = n_edges // _NW
    nsl = (n_nodes // _NS) // 8 * 8
    tail = n_nodes - _NS * nsl
    mesh = plsc.VectorSubcoreMesh(core_axis_name="c", subcore_axis_name="s")

    @functools.partial(
        pl.kernel,
        out_type=jax.ShapeDtypeStruct((_NC * n_nodes, d), jnp.float32),
        mesh=mesh,
        scratch_types=[
            pltpu.VMEM_SHARED((n_nodes, d), jnp.float32),
            pltpu.VMEM((ch,), jnp.int32),
            pltpu.VMEM((nsub, ks), jnp.int32),
            pltpu.VMEM((ch,), jnp.float32),
        ] + [pltpu.VMEM((ks, d), jnp.float32) for _ in range(nbuf)]
          + [pltpu.SemaphoreType.DMA for _ in range(2 * nbuf + 1)],
        compiler_params=pltpu.CompilerParams(needs_layout_passes=False),
    )
    def edge(table_hbm, flat_hbm, dst_hbm, inv_hbm, zero_hbm, out_hbm,
             agg_sh, idxb, dst2, invb, *bufs_and_sems):
        si = lax.axis_index("s")
        ci = lax.axis_index("c")
        wid = ci * _NS + si
        rowsb = bufs_and_sems[:nbuf]
        gsem = bufs_and_sems[nbuf:2 * nbuf]
        ssem = bufs_and_sems[2 * nbuf:3 * nbuf]
        hsem = bufs_and_sems[3 * nbuf]

        pltpu.sync_copy(zero_hbm.at[pl.ds(si * nsl, nsl)],
                        agg_sh.at[pl.ds(si * nsl, nsl)])
        if tail:
            @pl.when(si == _NS - 1)
            def _init_tail():
                pltpu.sync_copy(zero_hbm.at[pl.ds(_NS * nsl, tail)],
                                agg_sh.at[pl.ds(_NS * nsl, tail)])
        plsc.subcore_barrier()

        def scale(rb, off):
            def edge_body(e, _e):
                iv = plsc.load_gather(
                    invb, [jnp.full((_LANES,), off + e, jnp.int32)])
                for cc in range(d // _LANES):
                    sl = pl.ds(cc * _LANES, _LANES)
                    rb[e, sl] = rb[e, sl] * iv
                return _e
            lax.fori_loop(0, ks, edge_body, None, unroll=4)

        def chunk_body(c, _):
            base = wid * e_w + c * ch
            h0 = pltpu.async_copy(flat_hbm.at[pl.ds(base, ch)], idxb, hsem)
            h1 = pltpu.async_copy(inv_hbm.at[pl.ds(base, ch)], invb, hsem)
            h2 = pltpu.async_copy(
                dst_hbm.at[pl.ds(pl.multiple_of(base // ks, 8), nsub)], dst2,
                hsem)
            h0.wait()
            h1.wait()
            h2.wait()

            gd = [None] * nbuf
            sd = [None] * nbuf
            for k in range(nbuf - 1):
                gd[k] = pltpu.async_copy(
                    table_hbm.at[idxb.at[pl.ds(k * ks, ks)]],
                    rowsb[k], gsem[k])
            for k in range(nsub):
                b = k % nbuf
                if k + nbuf - 1 < nsub:
                    nb = (k + nbuf - 1) % nbuf
                    if sd[nb] is not None:
                        sd[nb].wait()
                        sd[nb] = None
                    gd[nb] = pltpu.async_copy(
                        table_hbm.at[idxb.at[pl.ds((k + nbuf - 1) * ks, ks)]],
                        rowsb[nb], gsem[nb])
                gd[b].wait()
                scale(rowsb[b], k * ks)
                sd[b] = pltpu.async_copy(rowsb[b], agg_sh.at[dst2.at[k]],
                                         ssem[b], add=True)
            for b in range(nbuf):
                if sd[b] is not None:
                    sd[b].wait()
            return _
        lax.fori_loop(0, e_w // ch, chunk_body, None)

        plsc.subcore_barrier()
        pltpu.sync_copy(agg_sh.at[pl.ds(si * nsl, nsl)],
                        out_hbm.at[pl.ds(ci * n_nodes + si * nsl, nsl)])
        if tail:
            @pl.when(si == _NS - 1)
            def _out_tail():
                pltpu.sync_copy(
                    agg_sh.at[pl.ds(_NS * nsl, tail)],
                    out_hbm.at[pl.ds(ci * n_nodes + _NS * nsl, tail)])

    return edge


def _mm_call(n_nodes, d, h, n_rels, fuse_agg):
    bn = 1000
    grid = (n_nodes // bn,)

    def body(*refs):
        if fuse_agg:
            a_ref, zin_ref, w_ref, b_ref, xt_ref, z_ref = refs
            t = jax.nn.relu(a_ref[0] + a_ref[1] + zin_ref[...])
        else:
            x_ref, w_ref, b_ref, xt_ref, z_ref = refs
            t = x_ref[...]
        for r in range(n_rels):
            xt_ref[r] = jnp.dot(t, w_ref[r], preferred_element_type=jnp.float32)
        z_ref[...] = (jnp.dot(t, w_ref[n_rels],
                              preferred_element_type=jnp.float32)
                      + b_ref[...])

    in_specs = []
    if fuse_agg:
        in_specs.append(pl.BlockSpec((_NC, bn, d), lambda i: (0, i, 0)))
        in_specs.append(pl.BlockSpec((bn, d), lambda i: (i, 0)))
    else:
        in_specs.append(pl.BlockSpec((bn, d), lambda i: (i, 0)))
    in_specs.append(pl.BlockSpec((n_rels + 1, d, h), lambda i: (0, 0, 0)))
    in_specs.append(pl.BlockSpec((1, h), lambda i: (0, 0)))

    return pl.pallas_call(
        body,
        grid=grid,
        in_specs=in_specs,
        out_specs=[
            pl.BlockSpec((n_rels, bn, h), lambda i: (0, i, 0)),
            pl.BlockSpec((bn, h), lambda i: (i, 0)),
        ],
        out_shape=[
            jax.ShapeDtypeStruct((n_rels, n_nodes, h), jnp.float32),
            jax.ShapeDtypeStruct((n_nodes, h), jnp.float32),
        ],
    )


def _final_call(n_nodes, d):
    bn = 1000
    grid = (n_nodes // bn,)

    def body(a_ref, z_ref, o_ref):
        o_ref[...] = a_ref[0] + a_ref[1] + z_ref[...]

    return pl.pallas_call(
        body,
        grid=grid,
        in_specs=[
            pl.BlockSpec((_NC, bn, d), lambda i: (0, i, 0)),
            pl.BlockSpec((bn, d), lambda i: (i, 0)),
        ],
        out_specs=pl.BlockSpec((bn, d), lambda i: (i, 0)),
        out_shape=jax.ShapeDtypeStruct((n_nodes, d), jnp.float32),
    )


def kernel(x, edge_index, edge_type, W1, root1, b1, W2, root2, b2):
    n_nodes, d = x.shape
    n_rels, _, h = W1.shape
    n_edges = edge_type.shape[0]
    o = W2.shape[2]

    src = edge_index[0]
    dst = edge_index[1]
    zeros_nd = jnp.zeros((n_nodes, d), jnp.float32)

    flat_idx, inv_d = _prep_call(n_nodes, n_rels, n_edges)(src, dst, edge_type)

    ch_total = _NW * 2560
    e_pad = -(-n_edges // ch_total) * ch_total
    pad = e_pad - n_edges
    spread = jnp.arange(pad, dtype=jnp.int32) % n_nodes
    flat_p = jnp.concatenate([flat_idx, spread])
    inv_p = jnp.concatenate([inv_d, jnp.zeros((pad,), jnp.float32)])
    dst_p = jnp.concatenate([dst, spread]).reshape(-1, 80)

    w1c = jnp.concatenate([W1, root1[None]], axis=0)
    w2c = jnp.concatenate([W2, root2[None]], axis=0)

    xt1, z1 = _mm_call(n_nodes, d, h, n_rels, fuse_agg=False)(
        x, w1c, b1.reshape(1, h))
    agg1 = _edge_call(n_nodes, h, e_pad)(
        xt1.reshape(n_rels * n_nodes, h), flat_p, dst_p, inv_p, zeros_nd)

    xt2, z2 = _mm_call(n_nodes, h, o, n_rels, fuse_agg=True)(
        agg1.reshape(_NC, n_nodes, h), z1, w2c, b2.reshape(1, o))
    agg2 = _edge_call(n_nodes, o, e_pad)(
        xt2.reshape(n_rels * n_nodes, o), flat_p, dst_p, inv_p, zeros_nd)

    return _final_call(n_nodes, o)(agg2.reshape(_NC, n_nodes, o), z2)


# speedup vs baseline: 2.0981x; 1.0164x over previous
import functools

import jax
import jax.numpy as jnp
from jax import lax
from jax.experimental import pallas as pl
from jax.experimental.pallas import tpu as pltpu
from jax.experimental.pallas import tpu_sc as plsc

_NC = 2
_NS = 16
_NW = _NC * _NS
_LANES = 16

_K = 80
_NCH = 25
_CH = _K * _NCH


def _prep_call(n_nodes, n_rels, n_edges):
    e_w = n_edges // _NW
    e_t = n_edges // _NS
    nrc = n_nodes * n_rels
    mesh = plsc.VectorSubcoreMesh(core_axis_name="c", subcore_axis_name="s")

    @functools.partial(
        pl.kernel,
        out_type=(
            jax.ShapeDtypeStruct((n_edges,), jnp.int32),
            jax.ShapeDtypeStruct((n_edges,), jnp.float32),
        ),
        mesh=mesh,
        scratch_types=[
            pltpu.VMEM_SHARED((nrc,), jnp.float32),
            pltpu.VMEM((_CH,), jnp.int32),
            pltpu.VMEM((_CH,), jnp.int32),
            pltpu.VMEM((_CH,), jnp.int32),
            pltpu.VMEM((_CH,), jnp.int32),
            pltpu.VMEM((_CH,), jnp.int32),
            pltpu.VMEM((_CH,), jnp.float32),
            pltpu.VMEM((_CH,), jnp.float32),
            pltpu.VMEM((_CH,), jnp.float32),
            pltpu.SemaphoreType.DMA,
        ],
        compiler_params=pltpu.CompilerParams(needs_layout_passes=False),
    )
    def prep(src_hbm, dst_hbm, et_hbm, flat_hbm, inv_hbm,
             cnt_sh, srcb, dstb, etb, comp1, flatb, onesb, valsb,
             invb, sem):
        si = lax.axis_index("s")
        ci = lax.axis_index("c")
        wid = ci * _NS + si

        def fill_ones(i, _):
            onesb[pl.ds(i * _LANES, _LANES)] = jnp.full((_LANES,), 1.0,
                                                        jnp.float32)
            return _
        lax.fori_loop(0, _CH // _LANES, fill_ones, None, unroll=4)

        def fill_zero(i, _):
            valsb[pl.ds(i * _LANES, _LANES)] = jnp.zeros((_LANES,),
                                                         jnp.float32)
            return _
        lax.fori_loop(0, _CH // _LANES, fill_zero, None)

        def zero_chunk(c, _):
            @pl.when(c % _NS == si)
            def _do():
                pltpu.sync_copy(valsb, cnt_sh.at[pl.ds(c * _CH, _CH)])
            return _
        lax.fori_loop(0, nrc // _CH, zero_chunk, None)
        plsc.subcore_barrier()

        def count_chunk(c, _):
            base = si * e_t + c * _CH
            pltpu.sync_copy(dst_hbm.at[pl.ds(base, _CH)], dstb)
            pltpu.sync_copy(et_hbm.at[pl.ds(base, _CH)], etb)

            def comp_grp(g, _g):
                sl = pl.ds(g * _LANES, _LANES)
                comp1[sl] = dstb[sl] * n_rels + etb[sl]
                return _g
            lax.fori_loop(0, _CH // _LANES, comp_grp, None, unroll=4)
            pltpu.sync_copy(onesb, cnt_sh.at[comp1], add=True)
            return _
        lax.fori_loop(0, e_t // _CH, count_chunk, None)
        plsc.subcore_barrier()

        def out_chunk(c, _):
            base = wid * e_w + c * _CH
            pltpu.sync_copy(src_hbm.at[pl.ds(base, _CH)], srcb)
            pltpu.sync_copy(dst_hbm.at[pl.ds(base, _CH)], dstb)
            pltpu.sync_copy(et_hbm.at[pl.ds(base, _CH)], etb)

            def idx_body(i, _i):
                sl = pl.ds(i * _LANES, _LANES)
                comp1[sl] = dstb[sl] * n_rels + etb[sl]
                flatb[sl] = etb[sl] * n_nodes + srcb[sl]
                return _i
            lax.fori_loop(0, _CH // _LANES, idx_body, None, unroll=4)

            pltpu.async_copy(cnt_sh.at[comp1], valsb, sem).wait()

            def inv_body(i, _i):
                sl = pl.ds(i * _LANES, _LANES)
                invb[sl] = 1.0 / jnp.maximum(valsb[sl], 1.0)
                return _i
            lax.fori_loop(0, _CH // _LANES, inv_body, None, unroll=4)

            pltpu.sync_copy(flatb, flat_hbm.at[pl.ds(base, _CH)])
            pltpu.sync_copy(invb, inv_hbm.at[pl.ds(base, _CH)])
            return _
        lax.fori_loop(0, e_w // _CH, out_chunk, None)

    return prep


def _edge_call(n_nodes, d, n_edges):
    ks = 160
    ch = 2560
    nsub = ch // ks
    nbuf = 2
    e_w = n_edges // _NW
    nsl = (n_nodes // _NS) // 8 * 8
    tail = n_nodes - _NS * nsl
    mesh = plsc.VectorSubcoreMesh(core_axis_name="c", subcore_axis_name="s")

    @functools.partial(
        pl.kernel,
        out_type=jax.ShapeDtypeStruct((_NC * n_nodes, d), jnp.float32),
        mesh=mesh,
        scratch_types=[
            pltpu.VMEM_SHARED((n_nodes, d), jnp.float32),
            pltpu.VMEM((ch,), jnp.int32),
            pltpu.VMEM((2 * nsub, ks // 2), jnp.int32),
            pltpu.VMEM((ch,), jnp.float32),
        ] + [pltpu.VMEM((ks, d), jnp.float32) for _ in range(nbuf)]
          + [pltpu.SemaphoreType.DMA for _ in range(2 * nbuf + 1)],
        compiler_params=pltpu.CompilerParams(needs_layout_passes=False),
    )
    def edge(table_hbm, flat_hbm, dst_hbm, inv_hbm, zero_hbm, out_hbm,
             agg_sh, idxb, dst2, invb, *bufs_and_sems):
        si = lax.axis_index("s")
        ci = lax.axis_index("c")
        wid = ci * _NS + si
        rowsb = bufs_and_sems[:nbuf]
        gsem = bufs_and_sems[nbuf:2 * nbuf]
        ssem = bufs_and_sems[2 * nbuf:3 * nbuf]
        hsem = bufs_and_sems[3 * nbuf]

        pltpu.sync_copy(zero_hbm.at[pl.ds(si * nsl, nsl)],
                        agg_sh.at[pl.ds(si * nsl, nsl)])
        if tail:
            @pl.when(si == _NS - 1)
            def _init_tail():
                pltpu.sync_copy(zero_hbm.at[pl.ds(_NS * nsl, tail)],
                                agg_sh.at[pl.ds(_NS * nsl, tail)])
        plsc.subcore_barrier()

        def scale(rb, off):
            def edge_body(e, _e):
                iv = plsc.load_gather(
                    invb, [jnp.full((_LANES,), off + e, jnp.int32)])
                for cc in range(d // _LANES):
                    sl = pl.ds(cc * _LANES, _LANES)
                    rb[e, sl] = rb[e, sl] * iv
                return _e
            lax.fori_loop(0, ks, edge_body, None, unroll=4)

        def chunk_body(c, _):
            base = wid * e_w + c * ch
            h0 = pltpu.async_copy(flat_hbm.at[pl.ds(base, ch)], idxb, hsem)
            h1 = pltpu.async_copy(inv_hbm.at[pl.ds(base, ch)], invb, hsem)
            h2 = pltpu.async_copy(
                dst_hbm.at[pl.ds(pl.multiple_of(base // (ks // 2), 8),
                                 2 * nsub)], dst2,
                hsem)
            h0.wait()
            h1.wait()
            h2.wait()

            gd = [None] * nbuf
            sd = [None] * nbuf
            for k in range(nbuf - 1):
                gd[k] = pltpu.async_copy(
                    table_hbm.at[idxb.at[pl.ds(k * ks, ks)]],
                    rowsb[k], gsem[k])
            for k in range(nsub):
                b = k % nbuf
                if k + nbuf - 1 < nsub:
                    nb = (k + nbuf - 1) % nbuf
                    if sd[nb] is not None:
                        sd[nb][0].wait()
                        sd[nb][1].wait()
                        sd[nb] = None
                    gd[nb] = pltpu.async_copy(
                        table_hbm.at[idxb.at[pl.ds((k + nbuf - 1) * ks, ks)]],
                        rowsb[nb], gsem[nb])
                gd[b].wait()
                scale(rowsb[b], k * ks)
                s0 = pltpu.async_copy(rowsb[b].at[pl.ds(0, ks // 2)],
                                      agg_sh.at[dst2.at[2 * k]],
                                      ssem[b], add=True)
                s1 = pltpu.async_copy(rowsb[b].at[pl.ds(ks // 2, ks // 2)],
                                      agg_sh.at[dst2.at[2 * k + 1]],
                                      ssem[b], add=True)
                sd[b] = (s0, s1)
            for b in range(nbuf):
                if sd[b] is not None:
                    sd[b][0].wait()
                    sd[b][1].wait()
            return _
        lax.fori_loop(0, e_w // ch, chunk_body, None)

        plsc.subcore_barrier()
        pltpu.sync_copy(agg_sh.at[pl.ds(si * nsl, nsl)],
                        out_hbm.at[pl.ds(ci * n_nodes + si * nsl, nsl)])
        if tail:
            @pl.when(si == _NS - 1)
            def _out_tail():
                pltpu.sync_copy(
                    agg_sh.at[pl.ds(_NS * nsl, tail)],
                    out_hbm.at[pl.ds(ci * n_nodes + _NS * nsl, tail)])

    return edge


def _mm_call(n_nodes, d, h, n_rels, fuse_agg):
    bn = 1000
    grid = (n_nodes // bn,)

    def body(*refs):
        if fuse_agg:
            a_ref, zin_ref, w_ref, b_ref, xt_ref, z_ref = refs
            t = jax.nn.relu(a_ref[0] + a_ref[1] + zin_ref[...])
        else:
            x_ref, w_ref, b_ref, xt_ref, z_ref = refs
            t = x_ref[...]
        for r in range(n_rels):
            xt_ref[r] = jnp.dot(t, w_ref[r], preferred_element_type=jnp.float32)
        z_ref[...] = (jnp.dot(t, w_ref[n_rels],
                              preferred_element_type=jnp.float32)
                      + b_ref[...])

    in_specs = []
    if fuse_agg:
        in_specs.append(pl.BlockSpec((_NC, bn, d), lambda i: (0, i, 0)))
        in_specs.append(pl.BlockSpec((bn, d), lambda i: (i, 0)))
    else:
        in_specs.append(pl.BlockSpec((bn, d), lambda i: (i, 0)))
    in_specs.append(pl.BlockSpec((n_rels + 1, d, h), lambda i: (0, 0, 0)))
    in_specs.append(pl.BlockSpec((1, h), lambda i: (0, 0)))

    return pl.pallas_call(
        body,
        grid=grid,
        in_specs=in_specs,
        out_specs=[
            pl.BlockSpec((n_rels, bn, h), lambda i: (0, i, 0)),
            pl.BlockSpec((bn, h), lambda i: (i, 0)),
        ],
        out_shape=[
            jax.ShapeDtypeStruct((n_rels, n_nodes, h), jnp.float32),
            jax.ShapeDtypeStruct((n_nodes, h), jnp.float32),
        ],
    )


def _final_call(n_nodes, d):
    bn = 1000
    grid = (n_nodes // bn,)

    def body(a_ref, z_ref, o_ref):
        o_ref[...] = a_ref[0] + a_ref[1] + z_ref[...]

    return pl.pallas_call(
        body,
        grid=grid,
        in_specs=[
            pl.BlockSpec((_NC, bn, d), lambda i: (0, i, 0)),
            pl.BlockSpec((bn, d), lambda i: (i, 0)),
        ],
        out_specs=pl.BlockSpec((bn, d), lambda i: (i, 0)),
        out_shape=jax.ShapeDtypeStruct((n_nodes, d), jnp.float32),
    )


def kernel(x, edge_index, edge_type, W1, root1, b1, W2, root2, b2):
    n_nodes, d = x.shape
    n_rels, _, h = W1.shape
    n_edges = edge_type.shape[0]
    o = W2.shape[2]

    src = edge_index[0]
    dst = edge_index[1]
    zeros_nd = jnp.zeros((n_nodes, d), jnp.float32)

    flat_idx, inv_d = _prep_call(n_nodes, n_rels, n_edges)(src, dst, edge_type)

    ch_total = _NW * 2560
    e_pad = -(-n_edges // ch_total) * ch_total
    pad = e_pad - n_edges
    spread = jnp.arange(pad, dtype=jnp.int32) % n_nodes
    flat_p = jnp.concatenate([flat_idx, spread])
    inv_p = jnp.concatenate([inv_d, jnp.zeros((pad,), jnp.float32)])
    dst_p = jnp.concatenate([dst, spread]).reshape(-1, 80)

    w1c = jnp.concatenate([W1, root1[None]], axis=0)
    w2c = jnp.concatenate([W2, root2[None]], axis=0)

    xt1, z1 = _mm_call(n_nodes, d, h, n_rels, fuse_agg=False)(
        x, w1c, b1.reshape(1, h))
    agg1 = _edge_call(n_nodes, h, e_pad)(
        xt1.reshape(n_rels * n_nodes, h), flat_p, dst_p, inv_p, zeros_nd)

    xt2, z2 = _mm_call(n_nodes, h, o, n_rels, fuse_agg=True)(
        agg1.reshape(_NC, n_nodes, h), z1, w2c, b2.reshape(1, o))
    agg2 = _edge_call(n_nodes, o, e_pad)(
        xt2.reshape(n_rels * n_nodes, o), flat_p, dst_p, inv_p, zeros_nd)

    return _final_call(n_nodes, o)(agg2.reshape(_NC, n_nodes, o), z2)
